# baseline (speedup 1.0000x reference)
import functools

import jax
import jax.numpy as jnp
from jax import lax
from jax.experimental import pallas as pl
from jax.experimental.pallas import tpu as pltpu
from jax.experimental.pallas import tpu_sc as plsc

N_NODES = 10000
D_FEAT = 128
N_CLS = 64
N_EDGES = 320000

NC = 2
NS = 16
NW = NC * NS
NP = 10240
RPS = NP // NS
CHUNK = 128
NCH = 79
TOT_CH = NW * NCH
EPAD = TOT_CH * CHUNK
K_C0 = 154
K_C1 = 2 * NCH - K_C0

_mesh = plsc.VectorSubcoreMesh(core_axis_name="c", subcore_axis_name="s")
_sc_params = pltpu.CompilerParams(use_tc_tiling_on_sc=False)


@functools.partial(
    pl.kernel,
    mesh=_mesh,
    out_type=jax.ShapeDtypeStruct((NC, NP, 16), jnp.float32),
    compiler_params=_sc_params,
    scratch_types=[
        pltpu.VMEM((NCH, CHUNK), jnp.int32),
        pltpu.VMEM((CHUNK, 16), jnp.float32),
        pltpu.VMEM_SHARED((NP, 16), jnp.float32),
    ],
)
def _sc_degree(dst_hbm, ones_hbm, zeros_hbm, out_hbm, idx_v, ones_v, cnt_sh):
    cid = lax.axis_index("c")
    sid = lax.axis_index("s")
    wid = cid * NS + sid
    pltpu.sync_copy(zeros_hbm, cnt_sh.at[pl.ds(sid * RPS, RPS)])
    pltpu.sync_copy(ones_hbm, ones_v)
    pltpu.sync_copy(dst_hbm.at[pl.ds(wid * NCH, NCH)], idx_v)
    plsc.subcore_barrier()

    @pl.loop(0, NCH)
    def _(g):
        pltpu.sync_copy(ones_v, cnt_sh.at[idx_v.at[g]], add=True)

    plsc.subcore_barrier()
    pltpu.sync_copy(cnt_sh.at[pl.ds(sid * RPS, RPS)],
                    out_hbm.at[cid, pl.ds(sid * RPS, RPS)])


PAGE = 64
NBUF = 4


@functools.partial(
    pl.kernel,
    mesh=_mesh,
    out_type=jax.ShapeDtypeStruct((NC, NP, N_CLS), jnp.float32),
    compiler_params=_sc_params,
    scratch_types=[
        pltpu.VMEM((PAGE, CHUNK), jnp.int32),
        pltpu.VMEM((PAGE, CHUNK), jnp.int32),
        pltpu.VMEM((NBUF, CHUNK, N_CLS), jnp.float32),
        pltpu.VMEM_SHARED((NP, N_CLS), jnp.float32),
        pltpu.SemaphoreType.DMA,
        pltpu.SemaphoreType.DMA,
        pltpu.SemaphoreType.DMA,
        pltpu.SemaphoreType.DMA,
    ],
)
def _sc_hop(z_hbm, src_hbm, dst_hbm, zeros_hbm, out_hbm,
            src_v, dst_v, ring, acc_sh, s0, s1, s2, s3):
    cid = lax.axis_index("c")
    sid = lax.axis_index("s")
    sems = (s0, s1, s2, s3)
    pltpu.sync_copy(zeros_hbm, acc_sh.at[pl.ds(sid * RPS, RPS)])

    def page_loop(kp, base):
        pltpu.sync_copy(src_hbm.at[pl.ds(base, kp)], src_v.at[pl.ds(0, kp)])
        pltpu.sync_copy(dst_hbm.at[pl.ds(base, kp)], dst_v.at[pl.ds(0, kp)])
        for j in range(min(NBUF, kp)):
            pltpu.async_copy(z_hbm.at[src_v.at[j]], ring.at[j], sems[j])
        grp = kp // NBUF
        rem = kp % NBUF
        if grp:
            @pl.loop(0, NBUF * grp, step=NBUF)
            def _(g):
                for j in range(NBUF):
                    pltpu.make_async_copy(
                        z_hbm.at[src_v.at[g + j]], ring.at[j], sems[j]).wait()
                    pltpu.sync_copy(ring.at[j], acc_sh.at[dst_v.at[g + j]],
                                    add=True)

                    @pl.when(g + j + NBUF < kp)
                    def _():
                        pltpu.async_copy(z_hbm.at[src_v.at[g + j + NBUF]],
                                         ring.at[j], sems[j])
        for j in range(rem):
            i = NBUF * grp + j
            pltpu.make_async_copy(z_hbm.at[src_v.at[i]], ring.at[j],
                                  sems[j]).wait()
            pltpu.sync_copy(ring.at[j], acc_sh.at[dst_v.at[i]], add=True)

    def edge_loop(k, base):
        off = 0
        while off < k:
            kp = min(PAGE, k - off)
            page_loop(kp, base + off)
            off += kp

    @pl.when(cid == 0)
    def _():
        edge_loop(K_C0, sid * K_C0)

    @pl.when(cid == 1)
    def _():
        edge_loop(K_C1, NS * K_C0 + sid * K_C1)

    plsc.subcore_barrier()
    pltpu.sync_copy(acc_sh.at[pl.ds(sid * RPS, RPS)],
                    out_hbm.at[cid, pl.ds(sid * RPS, RPS)])


NR = NP // 2
NCR = NP * 16 // 128


def _tc_project(x2, B2):
    blk = 1024

    def body(x_ref, w_ref, o_ref):
        o_ref[...] = lax.dot_general(
            x_ref[...], w_ref[...], (((1,), (0,)), ((), ())),
            preferred_element_type=jnp.float32)

    return pl.pallas_call(
        body,
        grid=(NR // blk,),
        in_specs=[pl.BlockSpec((blk, 2 * D_FEAT), lambda i: (i, 0)),
                  pl.BlockSpec((2 * D_FEAT, 128), lambda i: (0, 0))],
        out_specs=pl.BlockSpec((blk, 128), lambda i: (i, 0)),
        out_shape=jax.ShapeDtypeStruct((NR, 128), jnp.float32),
    )(x2, B2)


def _tc_prep(d128, y):
    blk = 1024

    def body(d_ref, y_ref, z_ref):
        z_ref[...] = lax.rsqrt(d_ref[...]) * y_ref[...]

    return pl.pallas_call(
        body,
        grid=(NR // blk,),
        in_specs=[pl.BlockSpec((blk, 128), lambda i: (i, 0)),
                  pl.BlockSpec((blk, 128), lambda i: (i, 0))],
        out_specs=pl.BlockSpec((blk, 128), lambda i: (i, 0)),
        out_shape=jax.ShapeDtypeStruct((NR, 128), jnp.float32),
    )(d128, y)


def _tc_combine(s_r, z1, d128):
    blk = 1024

    def body(s_ref, z_ref, d_ref, o_ref):
        o_ref[...] = (s_ref[0] + s_ref[1] + z_ref[...]) / d_ref[...]

    return pl.pallas_call(
        body,
        grid=(NR // blk,),
        in_specs=[pl.BlockSpec((NC, blk, 128), lambda i: (0, i, 0)),
                  pl.BlockSpec((blk, 128), lambda i: (i, 0)),
                  pl.BlockSpec((blk, 128), lambda i: (i, 0))],
        out_specs=pl.BlockSpec((blk, 128), lambda i: (i, 0)),
        out_shape=jax.ShapeDtypeStruct((NR, 128), jnp.float32),
    )(s_r, z1, d128)


def _tc_final(s_r, z2, d128, b128):
    blk = 1000

    def body(s_ref, z_ref, d_ref, b_ref, o_ref):
        u = lax.rsqrt(d_ref[...]) * (s_ref[0] + s_ref[1] + z_ref[...]) + b_ref[...]
        for h in (0, 1):
            v = u[:, h * 64:(h + 1) * 64]
            m = jnp.max(v, axis=1, keepdims=True)
            e = jnp.exp(v - m)
            lse = jnp.log(jnp.sum(e, axis=1, keepdims=True))
            o_ref[:, h * 64:(h + 1) * 64] = v - m - lse

    return pl.pallas_call(
        body,
        grid=(N_NODES // (2 * blk),),
        in_specs=[pl.BlockSpec((NC, blk, 128), lambda i: (0, i, 0)),
                  pl.BlockSpec((blk, 128), lambda i: (i, 0)),
                  pl.BlockSpec((blk, 128), lambda i: (i, 0)),
                  pl.BlockSpec((1, 128), lambda i: (0, 0))],
        out_specs=pl.BlockSpec((blk, 128), lambda i: (i, 0)),
        out_shape=jax.ShapeDtypeStruct((N_NODES // 2, 128), jnp.float32),
    )(s_r, z2, d128, b128)


def kernel(x, edge_index, W, b):
    src = edge_index[0].astype(jnp.int32)
    dst = edge_index[1].astype(jnp.int32)
    pad = jnp.full((EPAD - N_EDGES,), N_NODES, jnp.int32)
    src3 = jnp.concatenate([src, pad]).reshape(TOT_CH, CHUNK)
    dst3 = jnp.concatenate([dst, pad]).reshape(TOT_CH, CHUNK)
    x2 = jnp.pad(x, ((0, NP - N_NODES), (0, 0))).reshape(NR, 2 * D_FEAT)
    wt = W.T
    B2 = jnp.concatenate(
        [jnp.concatenate([wt, jnp.zeros_like(wt)], axis=1),
         jnp.concatenate([jnp.zeros_like(wt), wt], axis=1)], axis=0)
    b128 = jnp.concatenate([b, b]).reshape(1, 128)

    ones16 = jnp.ones((CHUNK, 16), jnp.float32)
    zeros16 = jnp.zeros((RPS, 16), jnp.float32)
    zeros64 = jnp.zeros((RPS, N_CLS), jnp.float32)

    cnt = _sc_degree(dst3, ones16, zeros16)
    deg = cnt[0, :, 0] + cnt[1, :, 0] + 1.0
    d128 = jnp.broadcast_to(deg.reshape(NR, 2, 1), (NR, 2, 64)).reshape(NR, 128)
    y = _tc_project(x2, B2)
    z1 = _tc_prep(d128, y)
    s1 = _sc_hop(z1.reshape(NP, N_CLS), src3, dst3, zeros64)
    z2 = _tc_combine(s1.reshape(NC, NR, 128), z1, d128)
    s2 = _sc_hop(z2.reshape(NP, N_CLS), src3, dst3, zeros64)
    o128 = _tc_final(s2.reshape(NC, NR, 128), z2, d128, b128)
    return o128.reshape(N_NODES, N_CLS)


# speedup vs baseline: 1.0551x; 1.0551x over previous
import functools

import jax
import jax.numpy as jnp
from jax import lax
from jax.experimental import pallas as pl
from jax.experimental.pallas import tpu as pltpu
from jax.experimental.pallas import tpu_sc as plsc

N_NODES = 10000
D_FEAT = 128
N_CLS = 64
N_EDGES = 320000

NC = 2
NS = 16
NW = NC * NS
NP = 10240
RPS = NP // NS
CHUNK = 128
NCH = 79
TOT_CH = NW * NCH
EPAD = TOT_CH * CHUNK
K_C0 = 152
K_C1 = 2 * NCH - K_C0

_mesh = plsc.VectorSubcoreMesh(core_axis_name="c", subcore_axis_name="s")
_sc_params = pltpu.CompilerParams(use_tc_tiling_on_sc=False)


@functools.partial(
    pl.kernel,
    mesh=_mesh,
    out_type=jax.ShapeDtypeStruct((NC, NP, 16), jnp.float32),
    compiler_params=_sc_params,
    scratch_types=[
        pltpu.VMEM((NCH, CHUNK), jnp.int32),
        pltpu.VMEM((CHUNK, 16), jnp.float32),
        pltpu.VMEM_SHARED((NP, 16), jnp.float32),
    ],
)
def _sc_degree(dst_hbm, ones_hbm, zeros_hbm, out_hbm, idx_v, ones_v, cnt_sh):
    cid = lax.axis_index("c")
    sid = lax.axis_index("s")
    wid = cid * NS + sid
    pltpu.sync_copy(zeros_hbm, cnt_sh.at[pl.ds(sid * RPS, RPS)])
    pltpu.sync_copy(ones_hbm, ones_v)
    pltpu.sync_copy(dst_hbm.at[pl.ds(wid * NCH, NCH)], idx_v)
    plsc.subcore_barrier()

    @pl.loop(0, NCH)
    def _(g):
        pltpu.sync_copy(ones_v, cnt_sh.at[idx_v.at[g]], add=True)

    plsc.subcore_barrier()
    pltpu.sync_copy(cnt_sh.at[pl.ds(sid * RPS, RPS)],
                    out_hbm.at[cid, pl.ds(sid * RPS, RPS)])


PAGE = 64
NBUF = 4


@functools.partial(
    pl.kernel,
    mesh=_mesh,
    out_type=jax.ShapeDtypeStruct((NC, NP, N_CLS), jnp.float32),
    compiler_params=_sc_params,
    scratch_types=[
        pltpu.VMEM((PAGE, CHUNK), jnp.int32),
        pltpu.VMEM((PAGE, CHUNK), jnp.int32),
        pltpu.VMEM((NBUF, CHUNK, N_CLS), jnp.float32),
        pltpu.VMEM_SHARED((NP, N_CLS), jnp.float32),
        pltpu.SemaphoreType.DMA,
        pltpu.SemaphoreType.DMA,
        pltpu.SemaphoreType.DMA,
        pltpu.SemaphoreType.DMA,
    ],
)
def _sc_hop(z_hbm, src_hbm, dst_hbm, zeros_hbm, out_hbm,
            src_v, dst_v, ring, acc_sh, s0, s1, s2, s3):
    cid = lax.axis_index("c")
    sid = lax.axis_index("s")
    sems = (s0, s1, s2, s3)
    pltpu.sync_copy(zeros_hbm, acc_sh.at[pl.ds(sid * RPS, RPS)])

    def page_loop(kp, base):
        pltpu.sync_copy(src_hbm.at[pl.ds(base, kp)], src_v.at[pl.ds(0, kp)])
        pltpu.sync_copy(dst_hbm.at[pl.ds(base, kp)], dst_v.at[pl.ds(0, kp)])
        for j in range(min(NBUF, kp)):
            pltpu.async_copy(z_hbm.at[src_v.at[j]], ring.at[j], sems[j])
        grp = kp // NBUF
        rem = kp % NBUF
        if grp:
            @pl.loop(0, NBUF * grp, step=NBUF)
            def _(g):
                for j in range(NBUF):
                    pltpu.make_async_copy(
                        z_hbm.at[src_v.at[g + j]], ring.at[j], sems[j]).wait()
                    pltpu.sync_copy(ring.at[j], acc_sh.at[dst_v.at[g + j]],
                                    add=True)

                    @pl.when(g + j + NBUF < kp)
                    def _():
                        pltpu.async_copy(z_hbm.at[src_v.at[g + j + NBUF]],
                                         ring.at[j], sems[j])
        for j in range(rem):
            i = NBUF * grp + j
            pltpu.make_async_copy(z_hbm.at[src_v.at[i]], ring.at[j],
                                  sems[j]).wait()
            pltpu.sync_copy(ring.at[j], acc_sh.at[dst_v.at[i]], add=True)

    def edge_loop(k, base):
        off = 0
        while off < k:
            kp = min(PAGE, k - off)
            page_loop(kp, base + off)
            off += kp

    @pl.when(cid == 0)
    def _():
        edge_loop(K_C0, sid * K_C0)

    @pl.when(cid == 1)
    def _():
        edge_loop(K_C1, NS * K_C0 + sid * K_C1)

    plsc.subcore_barrier()
    pltpu.sync_copy(acc_sh.at[pl.ds(sid * RPS, RPS)],
                    out_hbm.at[cid, pl.ds(sid * RPS, RPS)])


NR = NP // 2
NCR = NP * 16 // 128


def _tc_project(x2, B2):
    blk = 1024

    def body(x_ref, w_ref, o_ref):
        o_ref[...] = lax.dot_general(
            x_ref[...], w_ref[...], (((1,), (0,)), ((), ())),
            preferred_element_type=jnp.float32)

    return pl.pallas_call(
        body,
        grid=(NR // blk,),
        in_specs=[pl.BlockSpec((blk, 2 * D_FEAT), lambda i: (i, 0)),
                  pl.BlockSpec((2 * D_FEAT, 128), lambda i: (0, 0))],
        out_specs=pl.BlockSpec((blk, 128), lambda i: (i, 0)),
        out_shape=jax.ShapeDtypeStruct((NR, 128), jnp.float32),
    )(x2, B2)


def _tc_prep(d128, y):
    blk = 1024

    def body(d_ref, y_ref, z_ref):
        z_ref[...] = lax.rsqrt(d_ref[...]) * y_ref[...]

    return pl.pallas_call(
        body,
        grid=(NR // blk,),
        in_specs=[pl.BlockSpec((blk, 128), lambda i: (i, 0)),
                  pl.BlockSpec((blk, 128), lambda i: (i, 0))],
        out_specs=pl.BlockSpec((blk, 128), lambda i: (i, 0)),
        out_shape=jax.ShapeDtypeStruct((NR, 128), jnp.float32),
    )(d128, y)


def _tc_combine(s_r, z1, d128):
    blk = 1024

    def body(s_ref, z_ref, d_ref, o_ref):
        o_ref[...] = (s_ref[0] + s_ref[1] + z_ref[...]) / d_ref[...]

    return pl.pallas_call(
        body,
        grid=(NR // blk,),
        in_specs=[pl.BlockSpec((NC, blk, 128), lambda i: (0, i, 0)),
                  pl.BlockSpec((blk, 128), lambda i: (i, 0)),
                  pl.BlockSpec((blk, 128), lambda i: (i, 0))],
        out_specs=pl.BlockSpec((blk, 128), lambda i: (i, 0)),
        out_shape=jax.ShapeDtypeStruct((NR, 128), jnp.float32),
    )(s_r, z1, d128)


def _tc_final(s_r, z2, d128, b128):
    blk = 1000

    def body(s_ref, z_ref, d_ref, b_ref, o_ref):
        u = lax.rsqrt(d_ref[...]) * (s_ref[0] + s_ref[1] + z_ref[...]) + b_ref[...]
        for h in (0, 1):
            v = u[:, h * 64:(h + 1) * 64]
            m = jnp.max(v, axis=1, keepdims=True)
            e = jnp.exp(v - m)
            lse = jnp.log(jnp.sum(e, axis=1, keepdims=True))
            o_ref[:, h * 64:(h + 1) * 64] = v - m - lse

    return pl.pallas_call(
        body,
        grid=(N_NODES // (2 * blk),),
        in_specs=[pl.BlockSpec((NC, blk, 128), lambda i: (0, i, 0)),
                  pl.BlockSpec((blk, 128), lambda i: (i, 0)),
                  pl.BlockSpec((blk, 128), lambda i: (i, 0)),
                  pl.BlockSpec((1, 128), lambda i: (0, 0))],
        out_specs=pl.BlockSpec((blk, 128), lambda i: (i, 0)),
        out_shape=jax.ShapeDtypeStruct((N_NODES // 2, 128), jnp.float32),
    )(s_r, z2, d128, b128)


def kernel(x, edge_index, W, b):
    src = edge_index[0].astype(jnp.int32)
    dst = edge_index[1].astype(jnp.int32)
    pad = jnp.full((EPAD - N_EDGES,), N_NODES, jnp.int32)
    src3 = jnp.concatenate([src, pad]).reshape(TOT_CH, CHUNK)
    dst3 = jnp.concatenate([dst, pad]).reshape(TOT_CH, CHUNK)
    x2 = jnp.pad(x, ((0, NP - N_NODES), (0, 0))).reshape(NR, 2 * D_FEAT)
    wt = W.T
    B2 = jnp.concatenate(
        [jnp.concatenate([wt, jnp.zeros_like(wt)], axis=1),
         jnp.concatenate([jnp.zeros_like(wt), wt], axis=1)], axis=0)
    b128 = jnp.concatenate([b, b]).reshape(1, 128)

    ones16 = jnp.ones((CHUNK, 16), jnp.float32)
    zeros16 = jnp.zeros((RPS, 16), jnp.float32)
    zeros64 = jnp.zeros((RPS, N_CLS), jnp.float32)

    cnt = _sc_degree(dst3, ones16, zeros16)
    deg = cnt[0, :, 0] + cnt[1, :, 0] + 1.0
    d128 = jnp.broadcast_to(deg.reshape(NR, 2, 1), (NR, 2, 64)).reshape(NR, 128)
    y = _tc_project(x2, B2)
    z1 = _tc_prep(d128, y)
    s1 = _sc_hop(z1.reshape(NP, N_CLS), src3, dst3, zeros64)
    z2 = _tc_combine(s1.reshape(NC, NR, 128), z1, d128)
    s2 = _sc_hop(z2.reshape(NP, N_CLS), src3, dst3, zeros64)
    o128 = _tc_final(s2.reshape(NC, NR, 128), z2, d128, b128)
    return o128.reshape(N_NODES, N_CLS)
